# Initial kernel scaffold; baseline (speedup 1.0000x reference)
import jax
import jax.numpy as jnp
from jax.experimental import pallas as pl


def kernel(x, edge_index, edge_attr, edge_delta, edge_self, audio_node_mask, params):
    raise NotImplementedError("write your pallas kernel here")


# speedup vs baseline: 1.6468x; 1.6468x over previous
import functools

import jax
import jax.numpy as jnp
from jax import lax
from jax.experimental import pallas as pl
from jax.experimental.pallas import tpu as pltpu
from jax.experimental.pallas import tpu_sc as plsc

N = 10000
C = 128
H = 4
EDIM = 16
E = 160000
N_PAD = 10240
E_PAD = 163840
NW = 32
T_EDGE = E_PAD // NW
CH = 128
N_CH = T_EDGE // CH
CH_T = 512
ROWS_PER_TILE = N_PAD // 16
RB = 256

_BIG = 1 << 30


def _sc_params():
    return pltpu.CompilerParams(needs_layout_passes=False,
                                use_tc_tiling_on_sc=False)


def _node_call(body, out_heights, blocked, full):
    grid = (N_PAD // RB,)
    in_specs = ([pl.BlockSpec((a.shape[0], RB), lambda i: (0, i)) for a in blocked]
                + [pl.BlockSpec(a.shape, lambda i: (0,) * a.ndim) for a in full])
    out_specs = [pl.BlockSpec((h, RB), lambda i: (0, i)) for h in out_heights]
    out_shape = [jax.ShapeDtypeStruct((h, N_PAD), jnp.float32) for h in out_heights]
    f = pl.pallas_call(body, grid=grid, in_specs=in_specs,
                       out_specs=out_specs, out_shape=out_shape)
    return f(*blocked, *full)


def _dot(a, b):
    return jnp.dot(a, b, preferred_element_type=jnp.float32)


def _in_body(xa, xv, mf, WaT, WvT, vec, gf_o):
    a = _dot(WaT[...], xa[...]) + vec[:, 0:1]
    v = _dot(WvT[...], xv[...]) + vec[:, 1:2]
    g = jnp.where(mf[...] > 0.5, a, v)
    gf_o[...] = jnp.maximum(g * vec[:, 2:3] + vec[:, 3:4], 0.0)


def _ae_body(ea, Acat, out):
    out[...] = _dot(ea[...], Acat[...])


def _proj_body(x, WdT, WsT, WatT, pd_o, ps_o, att_o):
    xv = x[...]
    pd_o[...] = _dot(WdT[...], xv)
    ps_o[...] = _dot(WsT[...], xv)
    att_o[...] = _dot(WatT[...], xv)


def _projs_body(sT, d0, d1, WdT, WsT, WatT, vd, vs, vat, pd_o, ps_o, att_o):
    S = sT[...]
    sw = jnp.where((d0[...] + d1[...])[0:1, :] > 0.0, 1.0, 0.0)
    pd_o[...] = _dot(WdT[...], S) + vd[:, 0:1] * sw + vd[:, 1:2]
    ps_o[...] = _dot(WsT[...], S) + vs[:, 0:1] * sw + vs[:, 1:2]
    att_o[...] = _dot(WatT[...], S) + vat[:, 0:1] * sw + vat[:, 1:2]


def _comb_body(sa, da0, da1, sb, db0, db1, gf,
               W2aT, W2bT, vec, WdT, WsT, g1_o, pd_o, ps_o):
    swa = jnp.where((da0[...] + da1[...])[0:1, :] > 0.0, 1.0, 0.0)
    swb = jnp.where((db0[...] + db1[...])[0:1, :] > 0.0, 1.0, 0.0)
    A = (_dot(W2aT[...], sa[...]) + vec[:, 0:1] * swa
         + _dot(W2bT[...], sb[...]) + vec[:, 1:2] * swb
         + vec[:, 2:3] + gf[...])
    g1 = jnp.maximum(A * vec[:, 3:4] + vec[:, 4:5], 0.0)
    g1_o[...] = g1
    pd_o[...] = _dot(WdT[...], g1)
    ps_o[...] = _dot(WsT[...], g1)


def _etrans_body(sT, c0, c1, W2T, vec, WdT, WsT, pd_o, ps_o):
    cnt = (c0[...] + c1[...])[0:1, :]
    y = (_dot(W2T[...], sT[...]) + vec[:, 0:1] * cnt) / jnp.maximum(cnt, 1.0)
    pd_o[...] = _dot(WdT[...], y)
    ps_o[...] = _dot(WsT[...], y)


def _etransbn_body(sT, c0, c1, g1, W2T, vec, WdT, WsT, g2_o, pd_o, ps_o):
    cnt = (c0[...] + c1[...])[0:1, :]
    y = (_dot(W2T[...], sT[...]) + vec[:, 0:1] * cnt) / jnp.maximum(cnt, 1.0)
    g2 = jnp.maximum((y + g1[...]) * vec[:, 1:2] + vec[:, 2:3], 0.0)
    g2_o[...] = g2
    pd_o[...] = _dot(WdT[...], g2)
    ps_o[...] = _dot(WsT[...], g2)


def _final_body(sT, c0, c1, g2, W2T, vec, WfcT, out_o):
    cnt = (c0[...] + c1[...])[0:1, :]
    y = (_dot(W2T[...], sT[...]) + vec[:, 0:1] * cnt) / jnp.maximum(cnt, 1.0)
    g3 = y + g2[...]
    out_o[...] = _dot(WfcT[...], g3) + vec[:8, 1:2]


def _ae_call(ea_pad, Acat):
    grid = (E_PAD // 2048,)
    f = pl.pallas_call(
        _ae_body, grid=grid,
        in_specs=[pl.BlockSpec((2048, EDIM), lambda i: (i, 0)),
                  pl.BlockSpec((EDIM, 2 * H), lambda i: (0, 0))],
        out_specs=pl.BlockSpec((2048, 2 * H), lambda i: (i, 0)),
        out_shape=jax.ShapeDtypeStruct((E_PAD, 2 * H), jnp.float32))
    return f(ea_pad, Acat)


@functools.lru_cache(maxsize=None)
def _mesh():
    return plsc.VectorSubcoreMesh(core_axis_name="c", subcore_axis_name="s",
                                  num_cores=2, num_subcores=16)


def _worker_id():
    return lax.axis_index("s") * 2 + lax.axis_index("c")


def _zero_flat(ref, nwords):
    z = jnp.zeros((16,), jnp.float32)

    def zrow(i, _):
        ref[pl.ds(i * 16, 16)] = z
        return 0

    lax.fori_loop(0, nwords // 16, zrow, 0)


def _zero_rows16(ref, nrows):
    z = jnp.zeros((16,), jnp.float32)

    def zrow(i, _):
        ref[i, :] = z
        return 0

    lax.fori_loop(0, nrows, zrow, 0)


@functools.lru_cache(maxsize=None)
def _make_sc_att(lo, hi, use_self, ho):
    @functools.partial(
        pl.kernel,
        out_type=(jax.ShapeDtypeStruct((E_PAD * H,), jnp.float32),
                  jax.ShapeDtypeStruct((N_PAD, 16), jnp.float32),
                  jax.ShapeDtypeStruct((N_PAD, 16), jnp.float32)),
        mesh=_mesh(),
        compiler_params=_sc_params(),
        scratch_types=[
            pltpu.VMEM((8 * N_PAD,), jnp.float32),
            pltpu.VMEM((CH,), jnp.int32),
            pltpu.VMEM((CH,), jnp.int32),
            pltpu.VMEM((CH,), jnp.int32),
            pltpu.VMEM((CH,), jnp.int32),
            pltpu.VMEM((CH * 2 * H,), jnp.float32),
            pltpu.VMEM((CH * H,), jnp.float32),
            pltpu.VMEM((CH, 16), jnp.float32),
            pltpu.VMEM_SHARED((N_PAD, 16), jnp.float32),
        ],
    )
    def kern(src_h, dst_h, dl_h, se_h, att_h, ae_h,
             ex_h, den0_h, den1_h,
             attb, srcb, dstb, dlb, seb, aeb, exb, dens, den_sh):
        cid = lax.axis_index("c")
        sid = lax.axis_index("s")
        wid = _worker_id()
        _zero_rows16(dens, CH)
        for k in range(ROWS_PER_TILE // CH):
            pltpu.sync_copy(dens, den_sh.at[pl.ds(sid * ROWS_PER_TILE + k * CH, CH)])
        pltpu.sync_copy(att_h, attb)
        plsc.subcore_barrier()

        iota16 = lax.iota(jnp.int32, 16)

        def chunk(i, _):
            base = wid * T_EDGE + i * CH
            pltpu.sync_copy(src_h.at[pl.ds(base, CH)], srcb)
            pltpu.sync_copy(dst_h.at[pl.ds(base, CH)], dstb)
            pltpu.sync_copy(dl_h.at[pl.ds(base, CH)], dlb)
            pltpu.sync_copy(se_h.at[pl.ds(base, CH)], seb)
            pltpu.sync_copy(ae_h.at[pl.ds(base * 2 * H, CH * 2 * H)], aeb)

            def group(g, _):
                off = g * 16
                rows = iota16 + off
                s16 = srcb[pl.ds(off, 16)]
                d16 = dstb[pl.ds(off, 16)]
                dl16 = dlb[pl.ds(off, 16)]
                m = (dl16 >= lo) & (dl16 < hi)
                if use_self:
                    se16 = seb[pl.ds(off, 16)]
                    m = m | (se16 == 1)
                for h in range(H):
                    hv = jnp.full((16,), h, jnp.int32)
                    a_s = plsc.load_gather(attb, [s16 + h * N_PAD])
                    a_d = plsc.load_gather(attb, [d16 + (H + h) * N_PAD])
                    a_e = plsc.load_gather(aeb, [rows * (2 * H) + (ho + h)])
                    lg = a_s + a_d + a_e
                    lg = jnp.maximum(lg, 0.2 * lg)
                    exv = jnp.where(m, jnp.exp(lg), 0.0)
                    plsc.store_scatter(exb, [rows * H + h], exv)
                    plsc.store_scatter(dens, [rows, hv], exv)
                return 0

            lax.fori_loop(0, CH // 16, group, 0)
            pltpu.sync_copy(exb, ex_h.at[pl.ds(base * H, CH * H)])
            pltpu.sync_copy(dens, den_sh.at[dstb], add=True)
            return 0

        lax.fori_loop(0, N_CH, chunk, 0)
        plsc.subcore_barrier()
        rs = pl.ds(sid * ROWS_PER_TILE, ROWS_PER_TILE)

        @pl.when(cid == 0)
        def _():
            pltpu.sync_copy(den_sh.at[rs], den0_h.at[rs])

        @pl.when(cid == 1)
        def _():
            pltpu.sync_copy(den_sh.at[rs], den1_h.at[rs])

    return kern


@functools.lru_cache(maxsize=None)
def _make_sc_wgat():
    @functools.partial(
        pl.kernel,
        out_type=jax.ShapeDtypeStruct((E_PAD,), jnp.float32),
        mesh=_mesh(),
        compiler_params=_sc_params(),
        scratch_types=[
            pltpu.VMEM((CH,), jnp.int32),
            pltpu.VMEM((CH * H,), jnp.float32),
            pltpu.VMEM((CH, 16), jnp.float32),
            pltpu.VMEM((CH, 16), jnp.float32),
            pltpu.VMEM((CH,), jnp.float32),
        ],
    )
    def kern(dst_h, ex_h, den0_h, den1_h, w_h, dstb, exb, db0, db1, wb):
        wid = _worker_id()
        iota16 = lax.iota(jnp.int32, 16)

        def chunk(i, _):
            base = wid * T_EDGE + i * CH
            pltpu.sync_copy(dst_h.at[pl.ds(base, CH)], dstb)
            pltpu.sync_copy(ex_h.at[pl.ds(base * H, CH * H)], exb)
            pltpu.sync_copy(den0_h.at[dstb], db0)
            pltpu.sync_copy(den1_h.at[dstb], db1)

            def group(g, _):
                off = g * 16
                rows = iota16 + off
                acc = jnp.zeros((16,), jnp.float32)
                for h in range(H):
                    hv = jnp.full((16,), h, jnp.int32)
                    ev = plsc.load_gather(exb, [rows * H + h])
                    d0 = plsc.load_gather(db0, [rows, hv])
                    d1 = plsc.load_gather(db1, [rows, hv])
                    acc = acc + ev / (d0 + d1 + 1e-16)
                wb[pl.ds(off, 16)] = acc * 0.25
                return 0

            lax.fori_loop(0, CH // 16, group, 0)
            pltpu.sync_copy(wb, w_h.at[pl.ds(base, CH)])
            return 0

        lax.fori_loop(0, N_CH, chunk, 0)

    return kern


@functools.lru_cache(maxsize=None)
def _make_sc_cnt(lo, hi, use_self):
    @functools.partial(
        pl.kernel,
        out_type=(jax.ShapeDtypeStruct((E_PAD,), jnp.float32),
                  jax.ShapeDtypeStruct((N_PAD, 16), jnp.float32),
                  jax.ShapeDtypeStruct((N_PAD, 16), jnp.float32)),
        mesh=_mesh(),
        compiler_params=_sc_params(),
        scratch_types=[
            pltpu.VMEM((CH,), jnp.int32),
            pltpu.VMEM((CH,), jnp.int32),
            pltpu.VMEM((CH,), jnp.int32),
            pltpu.VMEM((CH,), jnp.float32),
            pltpu.VMEM((CH, 16), jnp.float32),
            pltpu.VMEM_SHARED((N_PAD, 16), jnp.float32),
        ],
    )
    def kern(dst_h, dl_h, se_h, w_h, cnt0_h, cnt1_h,
             dstb, dlb, seb, wb, dens, cnt_sh):
        cid = lax.axis_index("c")
        sid = lax.axis_index("s")
        wid = _worker_id()
        _zero_rows16(dens, CH)
        for k in range(ROWS_PER_TILE // CH):
            pltpu.sync_copy(dens, cnt_sh.at[pl.ds(sid * ROWS_PER_TILE + k * CH, CH)])
        plsc.subcore_barrier()
        iota16 = lax.iota(jnp.int32, 16)
        zero16i = jnp.zeros((16,), jnp.int32)

        def chunk(i, _):
            base = wid * T_EDGE + i * CH
            pltpu.sync_copy(dst_h.at[pl.ds(base, CH)], dstb)
            pltpu.sync_copy(dl_h.at[pl.ds(base, CH)], dlb)
            pltpu.sync_copy(se_h.at[pl.ds(base, CH)], seb)

            def group(g, _):
                off = g * 16
                rows = iota16 + off
                dl16 = dlb[pl.ds(off, 16)]
                m = (dl16 >= lo) & (dl16 < hi)
                if use_self:
                    m = m | (seb[pl.ds(off, 16)] == 1)
                wv = jnp.where(m, 1.0, 0.0)
                wb[pl.ds(off, 16)] = wv
                plsc.store_scatter(dens, [rows, zero16i], wv)
                return 0

            lax.fori_loop(0, CH // 16, group, 0)
            pltpu.sync_copy(wb, w_h.at[pl.ds(base, CH)])
            pltpu.sync_copy(dens, cnt_sh.at[dstb], add=True)
            return 0

        lax.fori_loop(0, N_CH, chunk, 0)
        plsc.subcore_barrier()
        rs = pl.ds(sid * ROWS_PER_TILE, ROWS_PER_TILE)

        @pl.when(cid == 0)
        def _():
            pltpu.sync_copy(cnt_sh.at[rs], cnt0_h.at[rs])

        @pl.when(cid == 1)
        def _():
            pltpu.sync_copy(cnt_sh.at[rs], cnt1_h.at[rs])

    return kern


@functools.lru_cache(maxsize=None)
def _make_sc_main():
    COLS = C // NW

    @functools.partial(
        pl.kernel,
        out_type=jax.ShapeDtypeStruct((C * N_PAD,), jnp.float32),
        mesh=_mesh(),
        compiler_params=_sc_params(),
        scratch_types=[
            pltpu.VMEM((COLS * N_PAD,), jnp.float32),
            pltpu.VMEM((COLS * N_PAD,), jnp.float32),
            pltpu.VMEM((COLS * N_PAD,), jnp.float32),
            pltpu.VMEM((CH_T,), jnp.int32),
            pltpu.VMEM((CH_T,), jnp.int32),
            pltpu.VMEM((CH_T,), jnp.float32),
            pltpu.VMEM((C,), jnp.float32),
        ],
    )
    def kern(src_h, dst_h, w_h, pdT_h, psT_h, c_h, sT_h,
             slabPd, slabPs, accT, srcb, dstb, wb, cbuf):
        wid = _worker_id()
        base_col = COLS * wid
        pltpu.sync_copy(pdT_h.at[pl.ds(base_col * N_PAD, COLS * N_PAD)], slabPd)
        pltpu.sync_copy(psT_h.at[pl.ds(base_col * N_PAD, COLS * N_PAD)], slabPs)
        pltpu.sync_copy(c_h, cbuf)
        _zero_flat(accT, COLS * N_PAD)
        ccs = [plsc.load_gather(cbuf, [jnp.full((16,), base_col + cl, jnp.int32)])
               for cl in range(COLS)]

        def chunk(i, _):
            eb = i * CH_T
            pltpu.sync_copy(src_h.at[pl.ds(eb, CH_T)], srcb)
            pltpu.sync_copy(dst_h.at[pl.ds(eb, CH_T)], dstb)
            pltpu.sync_copy(w_h.at[pl.ds(eb, CH_T)], wb)

            def group(g, _):
                off = g * 16
                s16 = srcb[pl.ds(off, 16)]
                d16 = dstb[pl.ds(off, 16)]
                w16 = wb[pl.ds(off, 16)]
                for cl in range(COLS):
                    pdv = plsc.load_gather(slabPd, [d16 + cl * N_PAD])
                    psv = plsc.load_gather(slabPs, [s16 + cl * N_PAD])
                    v = jnp.maximum(pdv + psv + ccs[cl], 0.0) * w16
                    plsc.addupdate_scatter(accT, [d16 + cl * N_PAD], v)
                return 0

            lax.fori_loop(0, CH_T // 16, group, 0)
            return 0

        lax.fori_loop(0, E_PAD // CH_T, chunk, 0)
        pltpu.sync_copy(accT, sT_h.at[pl.ds(base_col * N_PAD, COLS * N_PAD)])

    return kern


def _fold_lpp(p):
    s = p['bn1']['gamma'] / jnp.sqrt(p['bn1']['var'] + 1e-5)
    t = p['bn1']['beta'] - p['bn1']['mean'] * s
    Wd = (p['W1'][:C] - p['W1'][C:]) * s[None]
    Ws = p['W1'][C:] * s[None]
    c = p['b1'] * s + t
    return Wd, Ws, c


def _fold_gat(p):
    W = p['W'].reshape(C, H, C)
    A_src = jnp.einsum('chk,hk->ch', W, p['att_src'])
    A_dst = jnp.einsum('chk,hk->ch', W, p['att_dst'])
    A_edge = jnp.einsum('dhk,hk->dh', p['W_edge'].reshape(EDIM, H, C), p['att_edge'])
    Wd, Ws, c = _fold_lpp(p['nn'])
    Wat = jnp.concatenate([A_src, A_dst], axis=1)
    return Wd, Ws, c, Wat, A_edge


def _bn_affine(p):
    s = p['gamma'] / jnp.sqrt(p['var'] + 1e-5)
    t = p['beta'] - p['mean'] * s
    return s, t


M1 = (-_BIG, 1, False)
M2 = (1, 4, True)
M3 = (4, 8, True)
M4 = (8, 15, True)
M5 = (15, _BIG, False)


def kernel(x, edge_index, edge_attr, edge_delta, edge_self, audio_node_mask, params):
    f32 = jnp.float32
    src = edge_index[0].astype(jnp.int32)
    dst = edge_index[1].astype(jnp.int32)
    dl = edge_delta.astype(jnp.int32)
    se = edge_self.astype(jnp.int32)
    padN = jnp.full((E_PAD - E,), N_PAD - 1, jnp.int32)
    pad0 = jnp.zeros((E_PAD - E,), jnp.int32)
    src = jnp.concatenate([src, padN])
    dst = jnp.concatenate([dst, padN])
    dl = jnp.concatenate([dl, pad0])
    se = jnp.concatenate([se, pad0])
    ea_pad = jnp.zeros((E_PAD, EDIM), f32).at[:E].set(edge_attr)

    zcol = jnp.zeros((C, N_PAD - N), f32)
    xaT = jnp.concatenate([x[:, 0, :].T, zcol], axis=1)
    xvT = jnp.concatenate([x[:, 1, :].T, zcol], axis=1)
    mfT = jnp.concatenate([audio_node_mask.astype(f32),
                           jnp.zeros((N_PAD - N,), f32)])
    mfT = jnp.broadcast_to(mfT[None, :], (C, N_PAD))

    P = params
    s0, t0 = _bn_affine(P['batch_0'])
    vec_in = jnp.stack([P['layer_0_a']['b'], P['layer_0_v']['b'], s0, t0], axis=1)
    gf = _node_call(_in_body, [C], [xaT, xvT, mfT],
                    [P['layer_0_a']['W'].T, P['layer_0_v']['W'].T, vec_in])[0]

    g1_fold = _fold_gat(P['layer_1_1'])
    g2_fold = _fold_gat(P['layer_1_2'])
    Acat = jnp.concatenate([g1_fold[4], g2_fold[4]], axis=1)
    aeall = _ae_call(ea_pad, Acat).reshape(-1)

    sc_main = _make_sc_main()
    sc_wgat = _make_sc_wgat()

    def gat_stream(fold, p, masks, ho):
        Wd, Ws, c, Wat, _ = fold
        W2, b2, bias = p['nn']['W2'], p['nn']['b2'], p['bias']
        WdFT = _dot(W2, Wd).T
        WsFT = _dot(W2, Ws).T
        WatFT = _dot(W2, Wat).T
        vd = jnp.stack([b2 @ Wd, bias @ Wd], axis=1)
        vs = jnp.stack([b2 @ Ws, bias @ Ws], axis=1)
        vat = jnp.stack([b2 @ Wat, bias @ Wat], axis=1)
        pdT, psT, attT = _node_call(_proj_body, [C, C, 2 * H], [gf],
                                    [Wd.T, Ws.T, Wat.T])
        res = None
        for k, m in enumerate(masks):
            ex, d0, d1 = _make_sc_att(m[0], m[1], m[2], ho)(
                src, dst, dl, se, attT.reshape(-1), aeall)
            w = sc_wgat(dst, ex, d0, d1)
            sT = sc_main(src, dst, w, pdT.reshape(-1), psT.reshape(-1), c)
            sT = sT.reshape(C, N_PAD)
            res = (sT, d0.T, d1.T)
            if k + 1 < len(masks):
                pdT, psT, attT = _node_call(
                    _projs_body, [C, C, 2 * H],
                    [sT, res[1], res[2]],
                    [WdFT, WsFT, WatFT, vd, vs, vat])
        return res

    S11 = gat_stream(g1_fold, P['layer_1_1'], (M1, M2), 0)
    S12 = gat_stream(g2_fold, P['layer_1_2'], (M3, M4, M5), H)

    WdL2, WsL2, cL2 = _fold_lpp(P['layer_2'])
    WdL3, WsL3, cL3 = _fold_lpp(P['layer_3'])
    s1v, t1v = _bn_affine(P['batch_1'])
    vec_comb = jnp.stack([P['layer_1_1']['nn']['b2'], P['layer_1_2']['nn']['b2'],
                          P['layer_1_1']['bias'] + P['layer_1_2']['bias'],
                          s1v, t1v], axis=1)
    g1, pdT, psT = _node_call(
        _comb_body, [C, C, C],
        [S11[0], S11[1], S11[2], S12[0], S12[1], S12[2], gf],
        [P['layer_1_1']['nn']['W2'].T, P['layer_1_2']['nn']['W2'].T,
         vec_comb, WdL2.T, WsL2.T])

    W2L2, b2L2 = P['layer_2']['W2'], P['layer_2']['b2']
    W2L3, b2L3 = P['layer_3']['W2'], P['layer_3']['b2']

    def edge_conv(maskspec, pdT, psT, cvec):
        w, c0, c1 = _make_sc_cnt(*maskspec)(dst, dl, se)
        sT = sc_main(src, dst, w, pdT.reshape(-1), psT.reshape(-1), cvec)
        return sT.reshape(C, N_PAD), c0.T, c1.T

    sT, c0, c1 = edge_conv(M1, pdT, psT, cL2)
    pdT, psT = _node_call(_etrans_body, [C, C], [sT, c0, c1],
                          [W2L2.T, b2L2[:, None], WdL2.T, WsL2.T])
    sT, c0, c1 = edge_conv(M2, pdT, psT, cL2)
    s2v, t2v = _bn_affine(P['batch_2'])
    vec_e2 = jnp.stack([b2L2, s2v, t2v], axis=1)
    g2, pdT, psT = _node_call(_etransbn_body, [C, C, C], [sT, c0, c1, g1],
                              [W2L2.T, vec_e2, WdL3.T, WsL3.T])
    sT, c0, c1 = edge_conv(M1, pdT, psT, cL3)
    pdT, psT = _node_call(_etrans_body, [C, C], [sT, c0, c1],
                          [W2L3.T, b2L3[:, None], WdL3.T, WsL3.T])
    sT, c0, c1 = edge_conv(M2, pdT, psT, cL3)
    WfcT = jnp.zeros((8, C), f32).at[:2, :].set(P['fc']['W'].T)
    bfc = jnp.zeros((C,), f32).at[:2].set(P['fc']['b'])
    vec_f = jnp.stack([b2L3, bfc], axis=1)
    outT = _node_call(_final_body, [8], [sT, c0, c1, g2],
                      [W2L3.T, vec_f, WfcT])[0]
    return outT[:2, :N].T


# speedup vs baseline: 1.9449x; 1.1810x over previous
import functools

import jax
import jax.numpy as jnp
from jax import lax
from jax.experimental import pallas as pl
from jax.experimental.pallas import tpu as pltpu
from jax.experimental.pallas import tpu_sc as plsc

N = 10000
C = 128
H = 4
EDIM = 16
E = 160000
N_PAD = 10240
E_PAD = 163840
NW = 32
T_EDGE = E_PAD // NW
CH = 128
N_CH = T_EDGE // CH
CH_T = 2048
ROWS_PER_TILE = N_PAD // 16
RB = 256

_BIG = 1 << 30


def _sc_params():
    return pltpu.CompilerParams(needs_layout_passes=False,
                                use_tc_tiling_on_sc=False)


def _node_call(body, out_heights, blocked, full):
    grid = (N_PAD // RB,)
    in_specs = ([pl.BlockSpec((a.shape[0], RB), lambda i: (0, i)) for a in blocked]
                + [pl.BlockSpec(a.shape, lambda i: (0,) * a.ndim) for a in full])
    out_specs = [pl.BlockSpec((h, RB), lambda i: (0, i)) for h in out_heights]
    out_shape = [jax.ShapeDtypeStruct((h, N_PAD), jnp.float32) for h in out_heights]
    f = pl.pallas_call(body, grid=grid, in_specs=in_specs,
                       out_specs=out_specs, out_shape=out_shape)
    return f(*blocked, *full)


def _dot(a, b):
    return jnp.dot(a, b, preferred_element_type=jnp.float32)


def _in_body(xa, xv, mf, WaT, WvT, vec, gf_o):
    a = _dot(WaT[...], xa[...]) + vec[:, 0:1]
    v = _dot(WvT[...], xv[...]) + vec[:, 1:2]
    g = jnp.where(mf[...] > 0.5, a, v)
    gf_o[...] = jnp.maximum(g * vec[:, 2:3] + vec[:, 3:4], 0.0)


def _ae_body(ea, Acat, out):
    out[...] = _dot(ea[...], Acat[...])


def _proj_body(x, WdT, WsT, WatT, pd_o, ps_o, att_o):
    xv = x[...]
    pd_o[...] = _dot(WdT[...], xv)
    ps_o[...] = _dot(WsT[...], xv)
    att_o[...] = _dot(WatT[...], xv)


def _projs_body(sT, d0, d1, WdT, WsT, WatT, vd, vs, vat, pd_o, ps_o, att_o):
    S = sT[...]
    sw = jnp.where((d0[...] + d1[...])[0:1, :] > 0.0, 1.0, 0.0)
    pd_o[...] = _dot(WdT[...], S) + vd[:, 0:1] * sw + vd[:, 1:2]
    ps_o[...] = _dot(WsT[...], S) + vs[:, 0:1] * sw + vs[:, 1:2]
    att_o[...] = _dot(WatT[...], S) + vat[:, 0:1] * sw + vat[:, 1:2]


def _comb_body(sa, da0, da1, sb, db0, db1, gf,
               W2aT, W2bT, vec, WdT, WsT, g1_o, pd_o, ps_o):
    swa = jnp.where((da0[...] + da1[...])[0:1, :] > 0.0, 1.0, 0.0)
    swb = jnp.where((db0[...] + db1[...])[0:1, :] > 0.0, 1.0, 0.0)
    A = (_dot(W2aT[...], sa[...]) + vec[:, 0:1] * swa
         + _dot(W2bT[...], sb[...]) + vec[:, 1:2] * swb
         + vec[:, 2:3] + gf[...])
    g1 = jnp.maximum(A * vec[:, 3:4] + vec[:, 4:5], 0.0)
    g1_o[...] = g1
    pd_o[...] = _dot(WdT[...], g1)
    ps_o[...] = _dot(WsT[...], g1)


def _etrans_body(sT, c0, c1, W2T, vec, WdT, WsT, pd_o, ps_o):
    cnt = (c0[...] + c1[...])[0:1, :]
    y = (_dot(W2T[...], sT[...]) + vec[:, 0:1] * cnt) / jnp.maximum(cnt, 1.0)
    pd_o[...] = _dot(WdT[...], y)
    ps_o[...] = _dot(WsT[...], y)


def _etransbn_body(sT, c0, c1, g1, W2T, vec, WdT, WsT, g2_o, pd_o, ps_o):
    cnt = (c0[...] + c1[...])[0:1, :]
    y = (_dot(W2T[...], sT[...]) + vec[:, 0:1] * cnt) / jnp.maximum(cnt, 1.0)
    g2 = jnp.maximum((y + g1[...]) * vec[:, 1:2] + vec[:, 2:3], 0.0)
    g2_o[...] = g2
    pd_o[...] = _dot(WdT[...], g2)
    ps_o[...] = _dot(WsT[...], g2)


def _final_body(sT, c0, c1, g2, W2T, vec, WfcT, out_o):
    cnt = (c0[...] + c1[...])[0:1, :]
    y = (_dot(W2T[...], sT[...]) + vec[:, 0:1] * cnt) / jnp.maximum(cnt, 1.0)
    g3 = y + g2[...]
    out_o[...] = _dot(WfcT[...], g3) + vec[:8, 1:2]


def _ae_call(ea_pad, Acat):
    grid = (E_PAD // 2048,)
    f = pl.pallas_call(
        _ae_body, grid=grid,
        in_specs=[pl.BlockSpec((2048, EDIM), lambda i: (i, 0)),
                  pl.BlockSpec((EDIM, 2 * H), lambda i: (0, 0))],
        out_specs=pl.BlockSpec((2048, 2 * H), lambda i: (i, 0)),
        out_shape=jax.ShapeDtypeStruct((E_PAD, 2 * H), jnp.float32))
    return f(ea_pad, Acat)


@functools.lru_cache(maxsize=None)
def _mesh():
    return plsc.VectorSubcoreMesh(core_axis_name="c", subcore_axis_name="s",
                                  num_cores=2, num_subcores=16)


def _worker_id():
    return lax.axis_index("s") * 2 + lax.axis_index("c")


def _zero_flat(ref, nwords):
    z = jnp.zeros((16,), jnp.float32)

    def zrow(i, _):
        ref[pl.ds(i * 16, 16)] = z
        return 0

    lax.fori_loop(0, nwords // 16, zrow, 0)


def _zero_rows16(ref, nrows):
    z = jnp.zeros((16,), jnp.float32)

    def zrow(i, _):
        ref[i, :] = z
        return 0

    lax.fori_loop(0, nrows, zrow, 0)


@functools.lru_cache(maxsize=None)
def _make_sc_att(lo, hi, use_self, ho):
    @functools.partial(
        pl.kernel,
        out_type=(jax.ShapeDtypeStruct((E_PAD * H,), jnp.float32),
                  jax.ShapeDtypeStruct((N_PAD, 16), jnp.float32),
                  jax.ShapeDtypeStruct((N_PAD, 16), jnp.float32)),
        mesh=_mesh(),
        compiler_params=_sc_params(),
        scratch_types=[
            pltpu.VMEM((8 * N_PAD,), jnp.float32),
            pltpu.VMEM((CH,), jnp.int32),
            pltpu.VMEM((CH,), jnp.int32),
            pltpu.VMEM((CH,), jnp.int32),
            pltpu.VMEM((CH,), jnp.int32),
            pltpu.VMEM((CH * 2 * H,), jnp.float32),
            pltpu.VMEM((CH * H,), jnp.float32),
            pltpu.VMEM((CH, 16), jnp.float32),
            pltpu.VMEM_SHARED((N_PAD, 16), jnp.float32),
        ],
    )
    def kern(src_h, dst_h, dl_h, se_h, att_h, ae_h,
             ex_h, den0_h, den1_h,
             attb, srcb, dstb, dlb, seb, aeb, exb, dens, den_sh):
        cid = lax.axis_index("c")
        sid = lax.axis_index("s")
        wid = _worker_id()
        _zero_rows16(dens, CH)
        for k in range(ROWS_PER_TILE // CH):
            pltpu.sync_copy(dens, den_sh.at[pl.ds(sid * ROWS_PER_TILE + k * CH, CH)])
        pltpu.sync_copy(att_h, attb)
        plsc.subcore_barrier()

        iota16 = lax.iota(jnp.int32, 16)

        def chunk(i, _):
            base = wid * T_EDGE + i * CH
            pltpu.sync_copy(src_h.at[pl.ds(base, CH)], srcb)
            pltpu.sync_copy(dst_h.at[pl.ds(base, CH)], dstb)
            pltpu.sync_copy(dl_h.at[pl.ds(base, CH)], dlb)
            pltpu.sync_copy(se_h.at[pl.ds(base, CH)], seb)
            pltpu.sync_copy(ae_h.at[pl.ds(base * 2 * H, CH * 2 * H)], aeb)

            def group(g, _):
                off = g * 16
                rows = iota16 + off
                s16 = srcb[pl.ds(off, 16)]
                d16 = dstb[pl.ds(off, 16)]
                dl16 = dlb[pl.ds(off, 16)]
                m = (dl16 >= lo) & (dl16 < hi)
                if use_self:
                    se16 = seb[pl.ds(off, 16)]
                    m = m | (se16 == 1)
                for h in range(H):
                    hv = jnp.full((16,), h, jnp.int32)
                    a_s = plsc.load_gather(attb, [s16 + h * N_PAD])
                    a_d = plsc.load_gather(attb, [d16 + (H + h) * N_PAD])
                    a_e = plsc.load_gather(aeb, [rows * (2 * H) + (ho + h)])
                    lg = a_s + a_d + a_e
                    lg = jnp.maximum(lg, 0.2 * lg)
                    exv = jnp.where(m, jnp.exp(lg), 0.0)
                    plsc.store_scatter(exb, [rows * H + h], exv)
                    plsc.store_scatter(dens, [rows, hv], exv)
                return 0

            lax.fori_loop(0, CH // 16, group, 0)
            pltpu.sync_copy(exb, ex_h.at[pl.ds(base * H, CH * H)])
            pltpu.sync_copy(dens, den_sh.at[dstb], add=True)
            return 0

        lax.fori_loop(0, N_CH, chunk, 0)
        plsc.subcore_barrier()
        rs = pl.ds(sid * ROWS_PER_TILE, ROWS_PER_TILE)

        @pl.when(cid == 0)
        def _():
            pltpu.sync_copy(den_sh.at[rs], den0_h.at[rs])

        @pl.when(cid == 1)
        def _():
            pltpu.sync_copy(den_sh.at[rs], den1_h.at[rs])

    return kern


@functools.lru_cache(maxsize=None)
def _make_sc_wgat():
    @functools.partial(
        pl.kernel,
        out_type=jax.ShapeDtypeStruct((E_PAD,), jnp.float32),
        mesh=_mesh(),
        compiler_params=_sc_params(),
        scratch_types=[
            pltpu.VMEM((CH,), jnp.int32),
            pltpu.VMEM((CH * H,), jnp.float32),
            pltpu.VMEM((CH, 16), jnp.float32),
            pltpu.VMEM((CH, 16), jnp.float32),
            pltpu.VMEM((CH,), jnp.float32),
        ],
    )
    def kern(dst_h, ex_h, den0_h, den1_h, w_h, dstb, exb, db0, db1, wb):
        wid = _worker_id()
        iota16 = lax.iota(jnp.int32, 16)

        def chunk(i, _):
            base = wid * T_EDGE + i * CH
            pltpu.sync_copy(dst_h.at[pl.ds(base, CH)], dstb)
            pltpu.sync_copy(ex_h.at[pl.ds(base * H, CH * H)], exb)
            pltpu.sync_copy(den0_h.at[dstb], db0)
            pltpu.sync_copy(den1_h.at[dstb], db1)

            def group(g, _):
                off = g * 16
                rows = iota16 + off
                acc = jnp.zeros((16,), jnp.float32)
                for h in range(H):
                    hv = jnp.full((16,), h, jnp.int32)
                    ev = plsc.load_gather(exb, [rows * H + h])
                    d0 = plsc.load_gather(db0, [rows, hv])
                    d1 = plsc.load_gather(db1, [rows, hv])
                    acc = acc + ev / (d0 + d1 + 1e-16)
                wb[pl.ds(off, 16)] = acc * 0.25
                return 0

            lax.fori_loop(0, CH // 16, group, 0)
            pltpu.sync_copy(wb, w_h.at[pl.ds(base, CH)])
            return 0

        lax.fori_loop(0, N_CH, chunk, 0)

    return kern


@functools.lru_cache(maxsize=None)
def _make_sc_cnt(lo, hi, use_self):
    @functools.partial(
        pl.kernel,
        out_type=(jax.ShapeDtypeStruct((E_PAD,), jnp.float32),
                  jax.ShapeDtypeStruct((N_PAD, 16), jnp.float32),
                  jax.ShapeDtypeStruct((N_PAD, 16), jnp.float32)),
        mesh=_mesh(),
        compiler_params=_sc_params(),
        scratch_types=[
            pltpu.VMEM((CH,), jnp.int32),
            pltpu.VMEM((CH,), jnp.int32),
            pltpu.VMEM((CH,), jnp.int32),
            pltpu.VMEM((CH,), jnp.float32),
            pltpu.VMEM((CH, 16), jnp.float32),
            pltpu.VMEM_SHARED((N_PAD, 16), jnp.float32),
        ],
    )
    def kern(dst_h, dl_h, se_h, w_h, cnt0_h, cnt1_h,
             dstb, dlb, seb, wb, dens, cnt_sh):
        cid = lax.axis_index("c")
        sid = lax.axis_index("s")
        wid = _worker_id()
        _zero_rows16(dens, CH)
        for k in range(ROWS_PER_TILE // CH):
            pltpu.sync_copy(dens, cnt_sh.at[pl.ds(sid * ROWS_PER_TILE + k * CH, CH)])
        plsc.subcore_barrier()
        iota16 = lax.iota(jnp.int32, 16)
        zero16i = jnp.zeros((16,), jnp.int32)

        def chunk(i, _):
            base = wid * T_EDGE + i * CH
            pltpu.sync_copy(dst_h.at[pl.ds(base, CH)], dstb)
            pltpu.sync_copy(dl_h.at[pl.ds(base, CH)], dlb)
            pltpu.sync_copy(se_h.at[pl.ds(base, CH)], seb)

            def group(g, _):
                off = g * 16
                rows = iota16 + off
                dl16 = dlb[pl.ds(off, 16)]
                m = (dl16 >= lo) & (dl16 < hi)
                if use_self:
                    m = m | (seb[pl.ds(off, 16)] == 1)
                wv = jnp.where(m, 1.0, 0.0)
                wb[pl.ds(off, 16)] = wv
                plsc.store_scatter(dens, [rows, zero16i], wv)
                return 0

            lax.fori_loop(0, CH // 16, group, 0)
            pltpu.sync_copy(wb, w_h.at[pl.ds(base, CH)])
            pltpu.sync_copy(dens, cnt_sh.at[dstb], add=True)
            return 0

        lax.fori_loop(0, N_CH, chunk, 0)
        plsc.subcore_barrier()
        rs = pl.ds(sid * ROWS_PER_TILE, ROWS_PER_TILE)

        @pl.when(cid == 0)
        def _():
            pltpu.sync_copy(cnt_sh.at[rs], cnt0_h.at[rs])

        @pl.when(cid == 1)
        def _():
            pltpu.sync_copy(cnt_sh.at[rs], cnt1_h.at[rs])

    return kern


@functools.lru_cache(maxsize=None)
def _make_sc_main():
    COLS = C // NW

    @functools.partial(
        pl.kernel,
        out_type=jax.ShapeDtypeStruct((C * N_PAD,), jnp.float32),
        mesh=_mesh(),
        compiler_params=_sc_params(),
        scratch_types=[
            pltpu.VMEM((COLS * N_PAD,), jnp.float32),
            pltpu.VMEM((COLS * N_PAD,), jnp.float32),
            pltpu.VMEM((COLS * N_PAD,), jnp.float32),
            pltpu.VMEM((CH_T,), jnp.int32),
            pltpu.VMEM((CH_T,), jnp.int32),
            pltpu.VMEM((CH_T,), jnp.float32),
            pltpu.VMEM((C,), jnp.float32),
        ],
    )
    def kern(src_h, dst_h, w_h, pdT_h, psT_h, c_h, sT_h,
             slabPd, slabPs, accT, srcb, dstb, wb, cbuf):
        wid = _worker_id()
        base_col = COLS * wid
        pltpu.sync_copy(pdT_h.at[pl.ds(base_col * N_PAD, COLS * N_PAD)], slabPd)
        pltpu.sync_copy(psT_h.at[pl.ds(base_col * N_PAD, COLS * N_PAD)], slabPs)
        pltpu.sync_copy(c_h, cbuf)
        _zero_flat(accT, COLS * N_PAD)
        ccs = [plsc.load_gather(cbuf, [jnp.full((16,), base_col + cl, jnp.int32)])
               for cl in range(COLS)]

        def chunk(i, _):
            eb = i * CH_T
            pltpu.sync_copy(src_h.at[pl.ds(eb, CH_T)], srcb)
            pltpu.sync_copy(dst_h.at[pl.ds(eb, CH_T)], dstb)
            pltpu.sync_copy(w_h.at[pl.ds(eb, CH_T)], wb)

            def group(g, _):
                off = g * 16
                w16 = wb[pl.ds(off, 16)]

                @pl.when(jnp.max(w16, axis=0) > 0.0)
                def _():
                    s16 = srcb[pl.ds(off, 16)]
                    d16 = dstb[pl.ds(off, 16)]
                    for cl in range(COLS):
                        pdv = plsc.load_gather(slabPd, [d16 + cl * N_PAD])
                        psv = plsc.load_gather(slabPs, [s16 + cl * N_PAD])
                        v = jnp.maximum(pdv + psv + ccs[cl], 0.0) * w16
                        plsc.addupdate_scatter(accT, [d16 + cl * N_PAD], v)

                return 0

            lax.fori_loop(0, CH_T // 16, group, 0)
            return 0

        lax.fori_loop(0, E_PAD // CH_T, chunk, 0)
        pltpu.sync_copy(accT, sT_h.at[pl.ds(base_col * N_PAD, COLS * N_PAD)])

    return kern


def _fold_lpp(p):
    s = p['bn1']['gamma'] / jnp.sqrt(p['bn1']['var'] + 1e-5)
    t = p['bn1']['beta'] - p['bn1']['mean'] * s
    Wd = (p['W1'][:C] - p['W1'][C:]) * s[None]
    Ws = p['W1'][C:] * s[None]
    c = p['b1'] * s + t
    return Wd, Ws, c


def _fold_gat(p):
    W = p['W'].reshape(C, H, C)
    A_src = jnp.einsum('chk,hk->ch', W, p['att_src'])
    A_dst = jnp.einsum('chk,hk->ch', W, p['att_dst'])
    A_edge = jnp.einsum('dhk,hk->dh', p['W_edge'].reshape(EDIM, H, C), p['att_edge'])
    Wd, Ws, c = _fold_lpp(p['nn'])
    Wat = jnp.concatenate([A_src, A_dst], axis=1)
    return Wd, Ws, c, Wat, A_edge


def _bn_affine(p):
    s = p['gamma'] / jnp.sqrt(p['var'] + 1e-5)
    t = p['beta'] - p['mean'] * s
    return s, t


M1 = (-_BIG, 1, False)
M2 = (1, 4, True)
M3 = (4, 8, True)
M4 = (8, 15, True)
M5 = (15, _BIG, False)


def kernel(x, edge_index, edge_attr, edge_delta, edge_self, audio_node_mask, params):
    f32 = jnp.float32
    src = edge_index[0].astype(jnp.int32)
    dst = edge_index[1].astype(jnp.int32)
    dl = edge_delta.astype(jnp.int32)
    se = edge_self.astype(jnp.int32)
    padN = jnp.full((E_PAD - E,), N_PAD - 1, jnp.int32)
    pad0 = jnp.zeros((E_PAD - E,), jnp.int32)
    src = jnp.concatenate([src, padN])
    dst = jnp.concatenate([dst, padN])
    dl = jnp.concatenate([dl, pad0])
    se = jnp.concatenate([se, pad0])
    ea_pad = jnp.zeros((E_PAD, EDIM), f32).at[:E].set(edge_attr)

    zcol = jnp.zeros((C, N_PAD - N), f32)
    xaT = jnp.concatenate([x[:, 0, :].T, zcol], axis=1)
    xvT = jnp.concatenate([x[:, 1, :].T, zcol], axis=1)
    mfT = jnp.concatenate([audio_node_mask.astype(f32),
                           jnp.zeros((N_PAD - N,), f32)])
    mfT = jnp.broadcast_to(mfT[None, :], (C, N_PAD))

    P = params
    s0, t0 = _bn_affine(P['batch_0'])
    vec_in = jnp.stack([P['layer_0_a']['b'], P['layer_0_v']['b'], s0, t0], axis=1)
    gf = _node_call(_in_body, [C], [xaT, xvT, mfT],
                    [P['layer_0_a']['W'].T, P['layer_0_v']['W'].T, vec_in])[0]

    g1_fold = _fold_gat(P['layer_1_1'])
    g2_fold = _fold_gat(P['layer_1_2'])
    Acat = jnp.concatenate([g1_fold[4], g2_fold[4]], axis=1)
    aeall = _ae_call(ea_pad, Acat).reshape(-1)

    sc_main = _make_sc_main()
    sc_wgat = _make_sc_wgat()

    def gat_stream(fold, p, masks, ho):
        Wd, Ws, c, Wat, _ = fold
        W2, b2, bias = p['nn']['W2'], p['nn']['b2'], p['bias']
        WdFT = _dot(W2, Wd).T
        WsFT = _dot(W2, Ws).T
        WatFT = _dot(W2, Wat).T
        vd = jnp.stack([b2 @ Wd, bias @ Wd], axis=1)
        vs = jnp.stack([b2 @ Ws, bias @ Ws], axis=1)
        vat = jnp.stack([b2 @ Wat, bias @ Wat], axis=1)
        pdT, psT, attT = _node_call(_proj_body, [C, C, 2 * H], [gf],
                                    [Wd.T, Ws.T, Wat.T])
        res = None
        for k, m in enumerate(masks):
            ex, d0, d1 = _make_sc_att(m[0], m[1], m[2], ho)(
                src, dst, dl, se, attT.reshape(-1), aeall)
            w = sc_wgat(dst, ex, d0, d1)
            sT = sc_main(src, dst, w, pdT.reshape(-1), psT.reshape(-1), c)
            sT = sT.reshape(C, N_PAD)
            res = (sT, d0.T, d1.T)
            if k + 1 < len(masks):
                pdT, psT, attT = _node_call(
                    _projs_body, [C, C, 2 * H],
                    [sT, res[1], res[2]],
                    [WdFT, WsFT, WatFT, vd, vs, vat])
        return res

    S11 = gat_stream(g1_fold, P['layer_1_1'], (M1, M2), 0)
    S12 = gat_stream(g2_fold, P['layer_1_2'], (M3, M4, M5), H)

    WdL2, WsL2, cL2 = _fold_lpp(P['layer_2'])
    WdL3, WsL3, cL3 = _fold_lpp(P['layer_3'])
    s1v, t1v = _bn_affine(P['batch_1'])
    vec_comb = jnp.stack([P['layer_1_1']['nn']['b2'], P['layer_1_2']['nn']['b2'],
                          P['layer_1_1']['bias'] + P['layer_1_2']['bias'],
                          s1v, t1v], axis=1)
    g1, pdT, psT = _node_call(
        _comb_body, [C, C, C],
        [S11[0], S11[1], S11[2], S12[0], S12[1], S12[2], gf],
        [P['layer_1_1']['nn']['W2'].T, P['layer_1_2']['nn']['W2'].T,
         vec_comb, WdL2.T, WsL2.T])

    W2L2, b2L2 = P['layer_2']['W2'], P['layer_2']['b2']
    W2L3, b2L3 = P['layer_3']['W2'], P['layer_3']['b2']

    def edge_conv(maskspec, pdT, psT, cvec):
        w, c0, c1 = _make_sc_cnt(*maskspec)(dst, dl, se)
        sT = sc_main(src, dst, w, pdT.reshape(-1), psT.reshape(-1), cvec)
        return sT.reshape(C, N_PAD), c0.T, c1.T

    sT, c0, c1 = edge_conv(M1, pdT, psT, cL2)
    pdT, psT = _node_call(_etrans_body, [C, C], [sT, c0, c1],
                          [W2L2.T, b2L2[:, None], WdL2.T, WsL2.T])
    sT, c0, c1 = edge_conv(M2, pdT, psT, cL2)
    s2v, t2v = _bn_affine(P['batch_2'])
    vec_e2 = jnp.stack([b2L2, s2v, t2v], axis=1)
    g2, pdT, psT = _node_call(_etransbn_body, [C, C, C], [sT, c0, c1, g1],
                              [W2L2.T, vec_e2, WdL3.T, WsL3.T])
    sT, c0, c1 = edge_conv(M1, pdT, psT, cL3)
    pdT, psT = _node_call(_etrans_body, [C, C], [sT, c0, c1],
                          [W2L3.T, b2L3[:, None], WdL3.T, WsL3.T])
    sT, c0, c1 = edge_conv(M2, pdT, psT, cL3)
    WfcT = jnp.zeros((8, C), f32).at[:2, :].set(P['fc']['W'].T)
    bfc = jnp.zeros((C,), f32).at[:2].set(P['fc']['b'])
    vec_f = jnp.stack([b2L3, bfc], axis=1)
    outT = _node_call(_final_body, [8], [sT, c0, c1, g2],
                      [W2L3.T, vec_f, WfcT])[0]
    return outT[:2, :N].T


# speedup vs baseline: 4.8207x; 2.4786x over previous
import functools

import jax
import jax.numpy as jnp
from jax import lax
from jax.experimental import pallas as pl
from jax.experimental.pallas import tpu as pltpu
from jax.experimental.pallas import tpu_sc as plsc

N = 10000
C = 128
H = 4
EDIM = 16
E = 160000
N_PAD = 10240
E_PAD = 163840
NW = 32
T_EDGE = E_PAD // NW
CH = 128
N_CH = T_EDGE // CH
CH_T = 2048
ROWS_PER_TILE = N_PAD // 16
RB = 256

_BIG = 1 << 30


def _sc_params():
    return pltpu.CompilerParams(needs_layout_passes=False,
                                use_tc_tiling_on_sc=False)


def _node_call(body, out_heights, blocked, full):
    grid = (N_PAD // RB,)
    in_specs = ([pl.BlockSpec((a.shape[0], RB), lambda i: (0, i)) for a in blocked]
                + [pl.BlockSpec(a.shape, lambda i: (0,) * a.ndim) for a in full])
    out_specs = [pl.BlockSpec((h, RB), lambda i: (0, i)) for h in out_heights]
    out_shape = [jax.ShapeDtypeStruct((h, N_PAD), jnp.float32) for h in out_heights]
    f = pl.pallas_call(body, grid=grid, in_specs=in_specs,
                       out_specs=out_specs, out_shape=out_shape)
    return f(*blocked, *full)


def _dot(a, b):
    return jnp.dot(a, b, preferred_element_type=jnp.float32)


def _in_body(xa, xv, mf, WaT, WvT, vec, gf_o):
    a = _dot(WaT[...], xa[...]) + vec[:, 0:1]
    v = _dot(WvT[...], xv[...]) + vec[:, 1:2]
    g = jnp.where(mf[...] > 0.5, a, v)
    gf_o[...] = jnp.maximum(g * vec[:, 2:3] + vec[:, 3:4], 0.0)


def _ae_body(ea, Acat, out):
    out[...] = _dot(ea[...], Acat[...])


def _proj_body(x, WdT, WsT, WatT, pd_o, ps_o, att_o):
    xv = x[...]
    pd_o[...] = _dot(WdT[...], xv)
    ps_o[...] = _dot(WsT[...], xv)
    att_o[...] = _dot(WatT[...], xv)


def _projs_body(sT, d0, d1, WdT, WsT, WatT, vd, vs, vat, pd_o, ps_o, att_o):
    S = sT[...]
    sw = jnp.where((d0[...] + d1[...])[0:1, :] > 0.0, 1.0, 0.0)
    pd_o[...] = _dot(WdT[...], S) + vd[:, 0:1] * sw + vd[:, 1:2]
    ps_o[...] = _dot(WsT[...], S) + vs[:, 0:1] * sw + vs[:, 1:2]
    att_o[...] = _dot(WatT[...], S) + vat[:, 0:1] * sw + vat[:, 1:2]


def _comb_body(sa, da0, da1, sb, db0, db1, gf,
               W2aT, W2bT, vec, WdT, WsT, g1_o, pd_o, ps_o):
    swa = jnp.where((da0[...] + da1[...])[0:1, :] > 0.0, 1.0, 0.0)
    swb = jnp.where((db0[...] + db1[...])[0:1, :] > 0.0, 1.0, 0.0)
    A = (_dot(W2aT[...], sa[...]) + vec[:, 0:1] * swa
         + _dot(W2bT[...], sb[...]) + vec[:, 1:2] * swb
         + vec[:, 2:3] + gf[...])
    g1 = jnp.maximum(A * vec[:, 3:4] + vec[:, 4:5], 0.0)
    g1_o[...] = g1
    pd_o[...] = _dot(WdT[...], g1)
    ps_o[...] = _dot(WsT[...], g1)


def _etrans_body(sT, c0, c1, W2T, vec, WdT, WsT, pd_o, ps_o):
    cnt = (c0[...] + c1[...])[0:1, :]
    y = (_dot(W2T[...], sT[...]) + vec[:, 0:1] * cnt) / jnp.maximum(cnt, 1.0)
    pd_o[...] = _dot(WdT[...], y)
    ps_o[...] = _dot(WsT[...], y)


def _etransbn_body(sT, c0, c1, g1, W2T, vec, WdT, WsT, g2_o, pd_o, ps_o):
    cnt = (c0[...] + c1[...])[0:1, :]
    y = (_dot(W2T[...], sT[...]) + vec[:, 0:1] * cnt) / jnp.maximum(cnt, 1.0)
    g2 = jnp.maximum((y + g1[...]) * vec[:, 1:2] + vec[:, 2:3], 0.0)
    g2_o[...] = g2
    pd_o[...] = _dot(WdT[...], g2)
    ps_o[...] = _dot(WsT[...], g2)


def _final_body(sT, c0, c1, g2, W2T, vec, WfcT, out_o):
    cnt = (c0[...] + c1[...])[0:1, :]
    y = (_dot(W2T[...], sT[...]) + vec[:, 0:1] * cnt) / jnp.maximum(cnt, 1.0)
    g3 = y + g2[...]
    out_o[...] = _dot(WfcT[...], g3) + vec[:8, 1:2]


def _ae_call(ea_pad, Acat):
    grid = (E_PAD // 2048,)
    f = pl.pallas_call(
        _ae_body, grid=grid,
        in_specs=[pl.BlockSpec((2048, EDIM), lambda i: (i, 0)),
                  pl.BlockSpec((EDIM, 2 * H), lambda i: (0, 0))],
        out_specs=pl.BlockSpec((2048, 2 * H), lambda i: (i, 0)),
        out_shape=jax.ShapeDtypeStruct((E_PAD, 2 * H), jnp.float32))
    return f(ea_pad, Acat)


@functools.lru_cache(maxsize=None)
def _mesh():
    return plsc.VectorSubcoreMesh(core_axis_name="c", subcore_axis_name="s",
                                  num_cores=2, num_subcores=16)


def _worker_id():
    return lax.axis_index("s") * 2 + lax.axis_index("c")


def _zero_flat(ref, nwords):
    z = jnp.zeros((16,), jnp.float32)

    def zrow(i, _):
        ref[pl.ds(i * 16, 16)] = z
        return 0

    lax.fori_loop(0, nwords // 16, zrow, 0)


def _zero_rows16(ref, nrows):
    z = jnp.zeros((16,), jnp.float32)

    def zrow(i, _):
        ref[i, :] = z
        return 0

    lax.fori_loop(0, nrows, zrow, 0)


@functools.lru_cache(maxsize=None)
def _make_sc_att(lo, hi, use_self, ho):
    @functools.partial(
        pl.kernel,
        out_type=(jax.ShapeDtypeStruct((E_PAD * H,), jnp.float32),
                  jax.ShapeDtypeStruct((N_PAD, 16), jnp.float32),
                  jax.ShapeDtypeStruct((N_PAD, 16), jnp.float32)),
        mesh=_mesh(),
        compiler_params=_sc_params(),
        scratch_types=[
            pltpu.VMEM((8 * N_PAD,), jnp.float32),
            pltpu.VMEM((CH,), jnp.int32),
            pltpu.VMEM((CH,), jnp.int32),
            pltpu.VMEM((CH,), jnp.int32),
            pltpu.VMEM((CH,), jnp.int32),
            pltpu.VMEM((CH * 2 * H,), jnp.float32),
            pltpu.VMEM((CH * H,), jnp.float32),
            pltpu.VMEM((CH, 16), jnp.float32),
            pltpu.VMEM_SHARED((N_PAD, 16), jnp.float32),
        ],
    )
    def kern(src_h, dst_h, dl_h, se_h, att_h, ae_h,
             ex_h, den0_h, den1_h,
             attb, srcb, dstb, dlb, seb, aeb, exb, dens, den_sh):
        cid = lax.axis_index("c")
        sid = lax.axis_index("s")
        wid = _worker_id()
        _zero_rows16(dens, CH)
        for k in range(ROWS_PER_TILE // CH):
            pltpu.sync_copy(dens, den_sh.at[pl.ds(sid * ROWS_PER_TILE + k * CH, CH)])
        pltpu.sync_copy(att_h, attb)
        plsc.subcore_barrier()

        iota16 = lax.iota(jnp.int32, 16)

        def chunk(i, _):
            base = wid * T_EDGE + i * CH
            pltpu.sync_copy(src_h.at[pl.ds(base, CH)], srcb)
            pltpu.sync_copy(dst_h.at[pl.ds(base, CH)], dstb)
            pltpu.sync_copy(dl_h.at[pl.ds(base, CH)], dlb)
            pltpu.sync_copy(se_h.at[pl.ds(base, CH)], seb)
            pltpu.sync_copy(ae_h.at[pl.ds(base * 2 * H, CH * 2 * H)], aeb)

            def group(g, _):
                off = g * 16
                rows = iota16 + off
                s16 = srcb[pl.ds(off, 16)]
                d16 = dstb[pl.ds(off, 16)]
                dl16 = dlb[pl.ds(off, 16)]
                m = (dl16 >= lo) & (dl16 < hi)
                if use_self:
                    se16 = seb[pl.ds(off, 16)]
                    m = m | (se16 == 1)
                for h in range(H):
                    hv = jnp.full((16,), h, jnp.int32)
                    a_s = plsc.load_gather(attb, [s16 + h * N_PAD])
                    a_d = plsc.load_gather(attb, [d16 + (H + h) * N_PAD])
                    a_e = plsc.load_gather(aeb, [rows * (2 * H) + (ho + h)])
                    lg = a_s + a_d + a_e
                    lg = jnp.maximum(lg, 0.2 * lg)
                    exv = jnp.where(m, jnp.exp(lg), 0.0)
                    plsc.store_scatter(exb, [rows * H + h], exv)
                    plsc.store_scatter(dens, [rows, hv], exv)
                return 0

            lax.fori_loop(0, CH // 16, group, 0)
            pltpu.sync_copy(exb, ex_h.at[pl.ds(base * H, CH * H)])
            pltpu.sync_copy(dens, den_sh.at[dstb], add=True)
            return 0

        lax.fori_loop(0, N_CH, chunk, 0)
        plsc.subcore_barrier()
        rs = pl.ds(sid * ROWS_PER_TILE, ROWS_PER_TILE)

        @pl.when(cid == 0)
        def _():
            pltpu.sync_copy(den_sh.at[rs], den0_h.at[rs])

        @pl.when(cid == 1)
        def _():
            pltpu.sync_copy(den_sh.at[rs], den1_h.at[rs])

    return kern


@functools.lru_cache(maxsize=None)
def _make_sc_wgat():
    @functools.partial(
        pl.kernel,
        out_type=jax.ShapeDtypeStruct((E_PAD,), jnp.float32),
        mesh=_mesh(),
        compiler_params=_sc_params(),
        scratch_types=[
            pltpu.VMEM((CH,), jnp.int32),
            pltpu.VMEM((CH * H,), jnp.float32),
            pltpu.VMEM((CH, 16), jnp.float32),
            pltpu.VMEM((CH, 16), jnp.float32),
            pltpu.VMEM((CH,), jnp.float32),
        ],
    )
    def kern(dst_h, ex_h, den0_h, den1_h, w_h, dstb, exb, db0, db1, wb):
        wid = _worker_id()
        iota16 = lax.iota(jnp.int32, 16)

        def chunk(i, _):
            base = wid * T_EDGE + i * CH
            pltpu.sync_copy(dst_h.at[pl.ds(base, CH)], dstb)
            pltpu.sync_copy(ex_h.at[pl.ds(base * H, CH * H)], exb)
            pltpu.sync_copy(den0_h.at[dstb], db0)
            pltpu.sync_copy(den1_h.at[dstb], db1)

            def group(g, _):
                off = g * 16
                rows = iota16 + off
                acc = jnp.zeros((16,), jnp.float32)
                for h in range(H):
                    hv = jnp.full((16,), h, jnp.int32)
                    ev = plsc.load_gather(exb, [rows * H + h])
                    d0 = plsc.load_gather(db0, [rows, hv])
                    d1 = plsc.load_gather(db1, [rows, hv])
                    acc = acc + ev / (d0 + d1 + 1e-16)
                wb[pl.ds(off, 16)] = acc * 0.25
                return 0

            lax.fori_loop(0, CH // 16, group, 0)
            pltpu.sync_copy(wb, w_h.at[pl.ds(base, CH)])
            return 0

        lax.fori_loop(0, N_CH, chunk, 0)

    return kern


@functools.lru_cache(maxsize=None)
def _make_sc_cnt(lo, hi, use_self):
    @functools.partial(
        pl.kernel,
        out_type=(jax.ShapeDtypeStruct((E_PAD,), jnp.float32),
                  jax.ShapeDtypeStruct((N_PAD, 16), jnp.float32),
                  jax.ShapeDtypeStruct((N_PAD, 16), jnp.float32)),
        mesh=_mesh(),
        compiler_params=_sc_params(),
        scratch_types=[
            pltpu.VMEM((CH,), jnp.int32),
            pltpu.VMEM((CH,), jnp.int32),
            pltpu.VMEM((CH,), jnp.int32),
            pltpu.VMEM((CH,), jnp.float32),
            pltpu.VMEM((CH, 16), jnp.float32),
            pltpu.VMEM_SHARED((N_PAD, 16), jnp.float32),
        ],
    )
    def kern(dst_h, dl_h, se_h, w_h, cnt0_h, cnt1_h,
             dstb, dlb, seb, wb, dens, cnt_sh):
        cid = lax.axis_index("c")
        sid = lax.axis_index("s")
        wid = _worker_id()
        _zero_rows16(dens, CH)
        for k in range(ROWS_PER_TILE // CH):
            pltpu.sync_copy(dens, cnt_sh.at[pl.ds(sid * ROWS_PER_TILE + k * CH, CH)])
        plsc.subcore_barrier()
        iota16 = lax.iota(jnp.int32, 16)
        zero16i = jnp.zeros((16,), jnp.int32)

        def chunk(i, _):
            base = wid * T_EDGE + i * CH
            pltpu.sync_copy(dst_h.at[pl.ds(base, CH)], dstb)
            pltpu.sync_copy(dl_h.at[pl.ds(base, CH)], dlb)
            pltpu.sync_copy(se_h.at[pl.ds(base, CH)], seb)

            def group(g, _):
                off = g * 16
                rows = iota16 + off
                dl16 = dlb[pl.ds(off, 16)]
                m = (dl16 >= lo) & (dl16 < hi)
                if use_self:
                    m = m | (seb[pl.ds(off, 16)] == 1)
                wv = jnp.where(m, 1.0, 0.0)
                wb[pl.ds(off, 16)] = wv
                plsc.store_scatter(dens, [rows, zero16i], wv)
                return 0

            lax.fori_loop(0, CH // 16, group, 0)
            pltpu.sync_copy(wb, w_h.at[pl.ds(base, CH)])
            pltpu.sync_copy(dens, cnt_sh.at[dstb], add=True)
            return 0

        lax.fori_loop(0, N_CH, chunk, 0)
        plsc.subcore_barrier()
        rs = pl.ds(sid * ROWS_PER_TILE, ROWS_PER_TILE)

        @pl.when(cid == 0)
        def _():
            pltpu.sync_copy(cnt_sh.at[rs], cnt0_h.at[rs])

        @pl.when(cid == 1)
        def _():
            pltpu.sync_copy(cnt_sh.at[rs], cnt1_h.at[rs])

    return kern


@functools.lru_cache(maxsize=None)
def _make_sc_main():
    COLS = C // NW

    @functools.partial(
        pl.kernel,
        out_type=jax.ShapeDtypeStruct((C * N_PAD,), jnp.float32),
        mesh=_mesh(),
        compiler_params=_sc_params(),
        scratch_types=[
            pltpu.VMEM((COLS * N_PAD,), jnp.float32),
            pltpu.VMEM((COLS * N_PAD,), jnp.float32),
            pltpu.VMEM((COLS * N_PAD,), jnp.float32),
            pltpu.VMEM((CH_T,), jnp.int32),
            pltpu.VMEM((CH_T,), jnp.int32),
            pltpu.VMEM((CH_T,), jnp.float32),
            pltpu.VMEM((C,), jnp.float32),
        ],
    )
    def kern(src_h, dst_h, w_h, pdT_h, psT_h, c_h, sT_h,
             slabPd, slabPs, accT, srcb, dstb, wb, cbuf):
        wid = _worker_id()
        base_col = COLS * wid
        pltpu.sync_copy(pdT_h.at[pl.ds(base_col * N_PAD, COLS * N_PAD)], slabPd)
        pltpu.sync_copy(psT_h.at[pl.ds(base_col * N_PAD, COLS * N_PAD)], slabPs)
        pltpu.sync_copy(c_h, cbuf)
        _zero_flat(accT, COLS * N_PAD)
        ccs = [plsc.load_gather(cbuf, [jnp.full((16,), base_col + cl, jnp.int32)])
               for cl in range(COLS)]

        def chunk(i, _):
            eb = i * CH_T
            pltpu.sync_copy(src_h.at[pl.ds(eb, CH_T)], srcb)
            pltpu.sync_copy(dst_h.at[pl.ds(eb, CH_T)], dstb)
            pltpu.sync_copy(w_h.at[pl.ds(eb, CH_T)], wb)

            @functools.partial(plsc.parallel_loop, 0, CH_T // 16, unroll=4)
            def group(g):
                off = g * 16
                w16 = wb[pl.ds(off, 16)]

                @pl.when(jnp.max(w16, axis=0) > 0.0)
                def _():
                    s16 = srcb[pl.ds(off, 16)]
                    d16 = dstb[pl.ds(off, 16)]
                    for cl in range(COLS):
                        pdv = plsc.load_gather(slabPd, [d16 + cl * N_PAD])
                        psv = plsc.load_gather(slabPs, [s16 + cl * N_PAD])
                        v = jnp.maximum(pdv + psv + ccs[cl], 0.0) * w16
                        plsc.addupdate_scatter(accT, [d16 + cl * N_PAD], v)

            return 0

        lax.fori_loop(0, E_PAD // CH_T, chunk, 0)
        pltpu.sync_copy(accT, sT_h.at[pl.ds(base_col * N_PAD, COLS * N_PAD)])

    return kern


def _fold_lpp(p):
    s = p['bn1']['gamma'] / jnp.sqrt(p['bn1']['var'] + 1e-5)
    t = p['bn1']['beta'] - p['bn1']['mean'] * s
    Wd = (p['W1'][:C] - p['W1'][C:]) * s[None]
    Ws = p['W1'][C:] * s[None]
    c = p['b1'] * s + t
    return Wd, Ws, c


def _fold_gat(p):
    W = p['W'].reshape(C, H, C)
    A_src = jnp.einsum('chk,hk->ch', W, p['att_src'])
    A_dst = jnp.einsum('chk,hk->ch', W, p['att_dst'])
    A_edge = jnp.einsum('dhk,hk->dh', p['W_edge'].reshape(EDIM, H, C), p['att_edge'])
    Wd, Ws, c = _fold_lpp(p['nn'])
    Wat = jnp.concatenate([A_src, A_dst], axis=1)
    return Wd, Ws, c, Wat, A_edge


def _bn_affine(p):
    s = p['gamma'] / jnp.sqrt(p['var'] + 1e-5)
    t = p['beta'] - p['mean'] * s
    return s, t


M1 = (-_BIG, 1, False)
M2 = (1, 4, True)
M3 = (4, 8, True)
M4 = (8, 15, True)
M5 = (15, _BIG, False)


def kernel(x, edge_index, edge_attr, edge_delta, edge_self, audio_node_mask, params):
    f32 = jnp.float32
    src = edge_index[0].astype(jnp.int32)
    dst = edge_index[1].astype(jnp.int32)
    dl = edge_delta.astype(jnp.int32)
    se = edge_self.astype(jnp.int32)
    padN = jnp.full((E_PAD - E,), N_PAD - 1, jnp.int32)
    pad0 = jnp.zeros((E_PAD - E,), jnp.int32)
    src = jnp.concatenate([src, padN])
    dst = jnp.concatenate([dst, padN])
    dl = jnp.concatenate([dl, pad0])
    se = jnp.concatenate([se, pad0])
    ea_pad = jnp.zeros((E_PAD, EDIM), f32).at[:E].set(edge_attr)

    zcol = jnp.zeros((C, N_PAD - N), f32)
    xaT = jnp.concatenate([x[:, 0, :].T, zcol], axis=1)
    xvT = jnp.concatenate([x[:, 1, :].T, zcol], axis=1)
    mfT = jnp.concatenate([audio_node_mask.astype(f32),
                           jnp.zeros((N_PAD - N,), f32)])
    mfT = jnp.broadcast_to(mfT[None, :], (C, N_PAD))

    P = params
    s0, t0 = _bn_affine(P['batch_0'])
    vec_in = jnp.stack([P['layer_0_a']['b'], P['layer_0_v']['b'], s0, t0], axis=1)
    gf = _node_call(_in_body, [C], [xaT, xvT, mfT],
                    [P['layer_0_a']['W'].T, P['layer_0_v']['W'].T, vec_in])[0]

    g1_fold = _fold_gat(P['layer_1_1'])
    g2_fold = _fold_gat(P['layer_1_2'])
    Acat = jnp.concatenate([g1_fold[4], g2_fold[4]], axis=1)
    aeall = _ae_call(ea_pad, Acat).reshape(-1)

    sc_main = _make_sc_main()
    sc_wgat = _make_sc_wgat()

    def gat_stream(fold, p, masks, ho):
        Wd, Ws, c, Wat, _ = fold
        W2, b2, bias = p['nn']['W2'], p['nn']['b2'], p['bias']
        WdFT = _dot(W2, Wd).T
        WsFT = _dot(W2, Ws).T
        WatFT = _dot(W2, Wat).T
        vd = jnp.stack([b2 @ Wd, bias @ Wd], axis=1)
        vs = jnp.stack([b2 @ Ws, bias @ Ws], axis=1)
        vat = jnp.stack([b2 @ Wat, bias @ Wat], axis=1)
        pdT, psT, attT = _node_call(_proj_body, [C, C, 2 * H], [gf],
                                    [Wd.T, Ws.T, Wat.T])
        res = None
        for k, m in enumerate(masks):
            ex, d0, d1 = _make_sc_att(m[0], m[1], m[2], ho)(
                src, dst, dl, se, attT.reshape(-1), aeall)
            w = sc_wgat(dst, ex, d0, d1)
            sT = sc_main(src, dst, w, pdT.reshape(-1), psT.reshape(-1), c)
            sT = sT.reshape(C, N_PAD)
            res = (sT, d0.T, d1.T)
            if k + 1 < len(masks):
                pdT, psT, attT = _node_call(
                    _projs_body, [C, C, 2 * H],
                    [sT, res[1], res[2]],
                    [WdFT, WsFT, WatFT, vd, vs, vat])
        return res

    S11 = gat_stream(g1_fold, P['layer_1_1'], (M1, M2), 0)
    S12 = gat_stream(g2_fold, P['layer_1_2'], (M3, M4, M5), H)

    WdL2, WsL2, cL2 = _fold_lpp(P['layer_2'])
    WdL3, WsL3, cL3 = _fold_lpp(P['layer_3'])
    s1v, t1v = _bn_affine(P['batch_1'])
    vec_comb = jnp.stack([P['layer_1_1']['nn']['b2'], P['layer_1_2']['nn']['b2'],
                          P['layer_1_1']['bias'] + P['layer_1_2']['bias'],
                          s1v, t1v], axis=1)
    g1, pdT, psT = _node_call(
        _comb_body, [C, C, C],
        [S11[0], S11[1], S11[2], S12[0], S12[1], S12[2], gf],
        [P['layer_1_1']['nn']['W2'].T, P['layer_1_2']['nn']['W2'].T,
         vec_comb, WdL2.T, WsL2.T])

    W2L2, b2L2 = P['layer_2']['W2'], P['layer_2']['b2']
    W2L3, b2L3 = P['layer_3']['W2'], P['layer_3']['b2']

    def edge_conv(maskspec, pdT, psT, cvec):
        w, c0, c1 = _make_sc_cnt(*maskspec)(dst, dl, se)
        sT = sc_main(src, dst, w, pdT.reshape(-1), psT.reshape(-1), cvec)
        return sT.reshape(C, N_PAD), c0.T, c1.T

    sT, c0, c1 = edge_conv(M1, pdT, psT, cL2)
    pdT, psT = _node_call(_etrans_body, [C, C], [sT, c0, c1],
                          [W2L2.T, b2L2[:, None], WdL2.T, WsL2.T])
    sT, c0, c1 = edge_conv(M2, pdT, psT, cL2)
    s2v, t2v = _bn_affine(P['batch_2'])
    vec_e2 = jnp.stack([b2L2, s2v, t2v], axis=1)
    g2, pdT, psT = _node_call(_etransbn_body, [C, C, C], [sT, c0, c1, g1],
                              [W2L2.T, vec_e2, WdL3.T, WsL3.T])
    sT, c0, c1 = edge_conv(M1, pdT, psT, cL3)
    pdT, psT = _node_call(_etrans_body, [C, C], [sT, c0, c1],
                          [W2L3.T, b2L3[:, None], WdL3.T, WsL3.T])
    sT, c0, c1 = edge_conv(M2, pdT, psT, cL3)
    WfcT = jnp.zeros((8, C), f32).at[:2, :].set(P['fc']['W'].T)
    bfc = jnp.zeros((C,), f32).at[:2].set(P['fc']['b'])
    vec_f = jnp.stack([b2L3, bfc], axis=1)
    outT = _node_call(_final_body, [8], [sT, c0, c1, g2],
                      [W2L3.T, vec_f, WfcT])[0]
    return outT[:2, :N].T
